# baseline (speedup 1.0000x reference)
import jax
import jax.numpy as jnp
from jax import lax
from jax.experimental import pallas as pl
from jax.experimental.pallas import tpu as pltpu
from jax.experimental.pallas import tpu_sc as plsc

DIM = 32
BATCH = 16384
NC, NS, L = 2, 16, 16
NW = NC * NS
BPW = BATCH // NW
CHUNK = 128
NCH = BPW // CHUNK


def _sc_body(aidx_hbm, bidx_hbm, a_hbm, b_hbm, out_hbm,
             aidx_v, bidx_v, arows_v, brows_v, out_v, sem):
    wid = lax.axis_index("s") * NC + lax.axis_index("c")
    base = wid * BPW

    pltpu.sync_copy(aidx_hbm.at[pl.ds(base, BPW)], aidx_v)
    pltpu.sync_copy(bidx_hbm.at[pl.ds(base, BPW)], bidx_v)

    copies = []
    for j in range(NCH):
        sl = pl.ds(j * CHUNK, CHUNK)
        copies.append(pltpu.async_copy(a_hbm.at[aidx_v.at[sl]], arows_v.at[sl], sem))
        copies.append(pltpu.async_copy(b_hbm.at[bidx_v.at[sl]], brows_v.at[sl], sem))
    for c in copies:
        c.wait()

    iota = lax.iota(jnp.int32, L)

    def group(g, carry):
        row0 = pl.multiple_of(g * L, L)
        ridx = row0 + iota
        acc = jnp.zeros((L,), jnp.float32)
        for d in range(DIM):
            cidx = jnp.full((L,), d, jnp.int32)
            av = plsc.load_gather(arows_v, [ridx, cidx])
            bv = plsc.load_gather(brows_v, [ridx, cidx])
            acc = acc + av * bv
        out_v[pl.ds(row0, L)] = acc
        return carry

    lax.fori_loop(0, BPW // L, group, 0)

    pltpu.sync_copy(out_v, out_hbm.at[pl.ds(base, BPW)])


def kernel(aIdx, bIdx, A, B):
    k = pl.kernel(
        _sc_body,
        out_type=jax.ShapeDtypeStruct((BATCH,), jnp.float32),
        mesh=plsc.VectorSubcoreMesh(core_axis_name="c", subcore_axis_name="s"),
        compiler_params=pltpu.CompilerParams(
            needs_layout_passes=False, use_tc_tiling_on_sc=False),
        scratch_types=[
            pltpu.VMEM((BPW,), jnp.int32),
            pltpu.VMEM((BPW,), jnp.int32),
            pltpu.VMEM((BPW, DIM), jnp.float32),
            pltpu.VMEM((BPW, DIM), jnp.float32),
            pltpu.VMEM((BPW,), jnp.float32),
            pltpu.SemaphoreType.DMA,
        ],
    )
    return k(aIdx.astype(jnp.int32), bIdx.astype(jnp.int32), A, B)


# speedup vs baseline: 2.3015x; 2.3015x over previous
import jax
import jax.numpy as jnp
from jax import lax
from jax.experimental import pallas as pl
from jax.experimental.pallas import tpu as pltpu
from jax.experimental.pallas import tpu_sc as plsc

DIM = 32
SUB = 8
BATCH = 16384
NC, NS, L = 2, 16, 16
NW = NC * NS
BPW = BATCH // NW
CH = 32
NCH = BPW // CH


def _sc_body(aidx_hbm, bidx_hbm, a_hbm, b_hbm, out_hbm,
             aidx_v, bidx_v, abuf, bbuf, out_v, sema, semb):
    wid = lax.axis_index("s") * NC + lax.axis_index("c")
    base = wid * BPW

    pltpu.sync_copy(aidx_hbm.at[pl.ds(base, BPW)], aidx_v)
    pltpu.sync_copy(bidx_hbm.at[pl.ds(base, BPW)], bidx_v)

    iota = lax.iota(jnp.int32, L)

    def chunk(k, carry):
        coff = pl.multiple_of(k * CH, CH)
        copies = []
        raws = []
        for g in range(CH // L):
            sl = pl.ds(coff + g * L, L)
            raws.append((aidx_v[sl], bidx_v[sl]))
        for g, (araw, braw) in enumerate(raws):
            for j in range(L):
                i = g * L + j
                ta = lax.shift_right_logical(araw[j], 3)
                tb = lax.shift_right_logical(braw[j], 3)
                copies.append(
                    pltpu.async_copy(a_hbm.at[ta], abuf.at[i], sema))
                copies.append(
                    pltpu.async_copy(b_hbm.at[tb], bbuf.at[i], semb))
        for c in copies:
            c.wait()
        for g, (araw, braw) in enumerate(raws):
            acc = jnp.zeros((L,), jnp.float32)
            for j in range(L):
                i = g * L + j
                sa = lax.bitwise_and(araw[j], 7)
                sb = lax.bitwise_and(braw[j], 7)
                p = (abuf[i, sa, pl.ds(0, L)] * bbuf[i, sb, pl.ds(0, L)]
                     + abuf[i, sa, pl.ds(L, L)] * bbuf[i, sb, pl.ds(L, L)])
                acc = jnp.where(iota == j, jnp.sum(p), acc)
            out_v[pl.ds(coff + g * L, L)] = acc
        return carry

    lax.fori_loop(0, NCH, chunk, 0)

    pltpu.sync_copy(out_v, out_hbm.at[pl.ds(base, BPW)])


def kernel(aIdx, bIdx, A, B):
    num = A.shape[0]
    k = pl.kernel(
        _sc_body,
        out_type=jax.ShapeDtypeStruct((BATCH,), jnp.float32),
        mesh=plsc.VectorSubcoreMesh(core_axis_name="c", subcore_axis_name="s"),
        compiler_params=pltpu.CompilerParams(needs_layout_passes=False),
        scratch_types=[
            pltpu.VMEM((BPW,), jnp.int32),
            pltpu.VMEM((BPW,), jnp.int32),
            pltpu.VMEM((CH, SUB, DIM), jnp.float32),
            pltpu.VMEM((CH, SUB, DIM), jnp.float32),
            pltpu.VMEM((BPW,), jnp.float32),
            pltpu.SemaphoreType.DMA,
            pltpu.SemaphoreType.DMA,
        ],
    )
    a3 = A.reshape(num // SUB, SUB, DIM)
    b3 = B.reshape(num // SUB, SUB, DIM)
    return k(aIdx.astype(jnp.int32), bIdx.astype(jnp.int32), a3, b3)
